# baseline (speedup 1.0000x reference)
import functools

import jax
import jax.numpy as jnp
from jax import lax
from jax.experimental import pallas as pl
from jax.experimental.pallas import tpu as pltpu
from jax.experimental.pallas import tpu_sc as plsc

H = 128
EPS = 1e-05
BB = 128


def _sc_gather(table, idx_pad, n_pad):
    info = plsc.get_sparse_core_info()
    nw = info.num_cores * info.num_subcores
    per = n_pad // nw

    @functools.partial(
        pl.kernel,
        mesh=plsc.VectorSubcoreMesh(core_axis_name="c", subcore_axis_name="s"),
        out_type=jax.ShapeDtypeStruct((n_pad, H), jnp.float32),
        scratch_types=[
            pltpu.VMEM((per,), jnp.int32),
            pltpu.VMEM((per, H), jnp.float32),
            pltpu.SemaphoreType.DMA,
        ],
    )
    def gk(table_hbm, idx_hbm, out_hbm, idx_v, rows_v, sem):
        wid = lax.axis_index("s") * info.num_cores + lax.axis_index("c")
        base = wid * per
        pltpu.sync_copy(idx_hbm.at[pl.ds(base, per)], idx_v)
        pltpu.async_copy(table_hbm.at[idx_v], rows_v, sem).wait()
        pltpu.sync_copy(rows_v, out_hbm.at[pl.ds(base, per)])

    return gk(table, idx_pad)


def _pq_body(l, emb_ref, mask_ref, lnw_ref, lnb_ref, bias_ref, w_ref, p_ref, q_ref):
    cl = mask_ref.shape[0]
    e = emb_ref[0:cl, :]
    mu = jnp.mean(e, axis=1, keepdims=True)
    d = e - mu
    var = jnp.mean(d * d, axis=1, keepdims=True)
    ln = d * lax.rsqrt(var + EPS) * lnw_ref[:] + lnb_ref[:]
    m = mask_ref[:]
    lm = (ln * m).reshape(cl // l, l, H)
    s = jnp.sum(lm, axis=1)
    cnt = jnp.sum(m.reshape(cl // l, l, 1), axis=1)
    col = s / cnt
    dn = (((1,), (1,)), ((), ()))
    p_ref[:] = lax.dot_general(col, w_ref[:], dn,
                               precision=lax.Precision.HIGHEST,
                               preferred_element_type=jnp.float32)
    q_ref[:] = lax.dot_general(bias_ref[:], w_ref[:], dn,
                               precision=lax.Precision.HIGHEST,
                               preferred_element_type=jnp.float32)


def _sc_bc(x_pad, p_pad, q_pad, b, c):
    info = plsc.get_sparse_core_info()
    nw = info.num_cores * info.num_subcores
    pw = b // nw
    nspan = c // 8
    rem = c - nspan * 8

    @functools.partial(
        pl.kernel,
        mesh=plsc.VectorSubcoreMesh(core_axis_name="c", subcore_axis_name="s"),
        out_type=jax.ShapeDtypeStruct((b, c, H), jnp.float32),
        scratch_types=[
            pltpu.VMEM((pw, H), jnp.float32),
            pltpu.VMEM((c, H), jnp.float32),
            pltpu.VMEM((1, H), jnp.float32),
            pltpu.VMEM((c, H), jnp.float32),
            pltpu.VMEM((c, H), jnp.float32),
            pltpu.SemaphoreType.DMA,
            pltpu.SemaphoreType.DMA,
        ],
    )
    def bk(x_hbm, p_hbm, q_hbm, out_hbm, x_v, p_v, q_v, buf0, buf1, sem0, sem1):
        wid = lax.axis_index("s") * info.num_cores + lax.axis_index("c")
        base = wid * pw
        pltpu.sync_copy(x_hbm.at[pl.ds(base, pw)], x_v)
        pltpu.sync_copy(p_hbm, p_v)
        pltpu.sync_copy(q_hbm, q_v)

        bufs = (buf0, buf1)
        sems = (sem0, sem1)

        def panel_dmas(buf, pnl, sem):
            cps = []
            for k in range(nspan):
                cps.append(pltpu.make_async_copy(
                    buf.at[pl.ds(8 * k, 8)],
                    out_hbm.at[pnl].at[pl.ds(8 * k, 8)], sem))
            if rem:
                cps.append(pltpu.make_async_copy(
                    buf.at[pl.ds(8 * nspan, rem)],
                    out_hbm.at[pnl].at[pl.ds(8 * nspan, rem)], sem))
            return cps

        def fma_row(buf, cc, xs):
            for h in range(H // 16):
                sl = pl.ds(16 * h, 16)
                buf[cc, sl] = xs * p_v[cc, sl] + q_v[0, sl]

        def compute_panel(i, buf):
            def grp(g, _):
                off = pl.multiple_of(16 * g, 16)
                xv = x_v[i, pl.ds(off, 16)]
                for j in range(16):
                    fma_row(buf, 16 * g + j, xv[j])
                return 0
            lax.fori_loop(0, c // 16, grp, 0)
            xv = x_v[i, pl.ds(16 * (c // 16), 16)]
            for j in range(c - 16 * (c // 16)):
                fma_row(buf, 16 * (c // 16) + j, xv[j])

        def step(s, _):
            for r in range(2):
                i = 2 * s + r
                pnl = base + i

                @pl.when(s > 0)
                def _():
                    for cp in panel_dmas(bufs[r], pnl - 2, sems[r]):
                        cp.wait()
                compute_panel(i, bufs[r])
                for cp in panel_dmas(bufs[r], pnl, sems[r]):
                    cp.start()
            return 0

        lax.fori_loop(0, pw // 2, step, 0)
        for r in range(2):
            for cp in panel_dmas(bufs[r], base + pw - 2 + r, sems[r]):
                cp.wait()

    return bk(x_pad, p_pad, q_pad)


def kernel(x_num, num_col_input_ids, num_att_mask, word_emb, ln_w, ln_b, num_bias, align_W):
    b, c = x_num.shape
    l = num_col_input_ids.shape[1]
    cl = c * l
    n_pad = ((cl + 255) // 256) * 256

    ids = num_col_input_ids.reshape(cl).astype(jnp.int32)
    ids = jnp.pad(ids, (0, n_pad - cl))
    emb = _sc_gather(word_emb, ids, n_pad)

    mask = num_att_mask.reshape(cl, 1).astype(jnp.float32)
    p, q = pl.pallas_call(
        functools.partial(_pq_body, l),
        out_shape=(
            jax.ShapeDtypeStruct((c, H), jnp.float32),
            jax.ShapeDtypeStruct((1, H), jnp.float32),
        ),
    )(emb, mask, ln_w.reshape(1, H), ln_b.reshape(1, H),
      num_bias.reshape(1, H), align_W)

    x_pad = jnp.pad(x_num, ((0, 0), (0, H - c)))
    out = _sc_bc(x_pad, p, q, b, c)

    attention_mask = jnp.ones((b, c), dtype=jnp.float32)
    return (out, attention_mask)


# speedup vs baseline: 1.0118x; 1.0118x over previous
import functools

import jax
import jax.numpy as jnp
from jax import lax
from jax.experimental import pallas as pl
from jax.experimental.pallas import tpu as pltpu
from jax.experimental.pallas import tpu_sc as plsc

H = 128
EPS = 1e-05
BB = 128


def _sc_gather(table, idx_pad, n_pad):
    info = plsc.get_sparse_core_info()
    nw = info.num_cores * info.num_subcores
    per = n_pad // nw

    @functools.partial(
        pl.kernel,
        mesh=plsc.VectorSubcoreMesh(core_axis_name="c", subcore_axis_name="s"),
        out_type=jax.ShapeDtypeStruct((n_pad, H), jnp.float32),
        scratch_types=[
            pltpu.VMEM((per,), jnp.int32),
            pltpu.VMEM((per, H), jnp.float32),
            pltpu.SemaphoreType.DMA,
        ],
    )
    def gk(table_hbm, idx_hbm, out_hbm, idx_v, rows_v, sem):
        wid = lax.axis_index("s") * info.num_cores + lax.axis_index("c")
        base = wid * per
        pltpu.sync_copy(idx_hbm.at[pl.ds(base, per)], idx_v)
        pltpu.async_copy(table_hbm.at[idx_v], rows_v, sem).wait()
        pltpu.sync_copy(rows_v, out_hbm.at[pl.ds(base, per)])

    return gk(table, idx_pad)


def _pq_body(l, emb_ref, mask_ref, lnw_ref, lnb_ref, bias_ref, w_ref, p_ref, q_ref):
    cl = mask_ref.shape[0]
    e = emb_ref[0:cl, :]
    mu = jnp.mean(e, axis=1, keepdims=True)
    d = e - mu
    var = jnp.mean(d * d, axis=1, keepdims=True)
    ln = d * lax.rsqrt(var + EPS) * lnw_ref[:] + lnb_ref[:]
    m = mask_ref[:]
    lm = (ln * m).reshape(cl // l, l, H)
    s = jnp.sum(lm, axis=1)
    cnt = jnp.sum(m.reshape(cl // l, l, 1), axis=1)
    col = s / cnt
    dn = (((1,), (1,)), ((), ()))
    p_ref[:] = lax.dot_general(col, w_ref[:], dn,
                               precision=lax.Precision.HIGHEST,
                               preferred_element_type=jnp.float32)
    q_ref[:] = lax.dot_general(bias_ref[:], w_ref[:], dn,
                               precision=lax.Precision.HIGHEST,
                               preferred_element_type=jnp.float32)


def _sc_bc(x_pad, p_pad, q_pad, b, c):
    info = plsc.get_sparse_core_info()
    nw = info.num_cores * info.num_subcores
    pw = b // nw
    nspan = c // 8
    rem = c - nspan * 8

    @functools.partial(
        pl.kernel,
        mesh=plsc.VectorSubcoreMesh(core_axis_name="c", subcore_axis_name="s"),
        out_type=jax.ShapeDtypeStruct((b, c, H), jnp.float32),
        scratch_types=[
            pltpu.VMEM((pw, H), jnp.float32),
            pltpu.VMEM((c, H), jnp.float32),
            pltpu.VMEM((1, H), jnp.float32),
            pltpu.VMEM((c, H), jnp.float32),
            pltpu.VMEM((c, H), jnp.float32),
            pltpu.SemaphoreType.DMA,
            pltpu.SemaphoreType.DMA,
        ],
    )
    def bk(x_hbm, p_hbm, q_hbm, out_hbm, x_v, p_v, q_v, buf0, buf1, sem0, sem1):
        wid = lax.axis_index("s") * info.num_cores + lax.axis_index("c")
        base = wid * pw
        pltpu.sync_copy(x_hbm.at[pl.ds(base, pw)], x_v)
        pltpu.sync_copy(p_hbm, p_v)
        pltpu.sync_copy(q_hbm, q_v)

        bufs = (buf0, buf1)
        sems = (sem0, sem1)

        def panel_dmas(buf, pnl, sem):
            return [pltpu.make_async_copy(buf, out_hbm.at[pnl], sem)]

        def fma_row(buf, cc, xs):
            for h in range(H // 16):
                sl = pl.ds(16 * h, 16)
                buf[cc, sl] = xs * p_v[cc, sl] + q_v[0, sl]

        def compute_panel(i, buf):
            def grp(g, _):
                off = pl.multiple_of(16 * g, 16)
                xv = x_v[i, pl.ds(off, 16)]
                for j in range(16):
                    fma_row(buf, 16 * g + j, xv[j])
                return 0
            lax.fori_loop(0, c // 16, grp, 0)
            xv = x_v[i, pl.ds(16 * (c // 16), 16)]
            for j in range(c - 16 * (c // 16)):
                fma_row(buf, 16 * (c // 16) + j, xv[j])

        def step(s, _):
            for r in range(2):
                i = 2 * s + r
                pnl = base + i

                @pl.when(s > 0)
                def _():
                    for cp in panel_dmas(bufs[r], pnl - 2, sems[r]):
                        cp.wait()
                compute_panel(i, bufs[r])
                for cp in panel_dmas(bufs[r], pnl, sems[r]):
                    cp.start()
            return 0

        lax.fori_loop(0, pw // 2, step, 0)
        for r in range(2):
            for cp in panel_dmas(bufs[r], base + pw - 2 + r, sems[r]):
                cp.wait()

    return bk(x_pad, p_pad, q_pad)


def kernel(x_num, num_col_input_ids, num_att_mask, word_emb, ln_w, ln_b, num_bias, align_W):
    b, c = x_num.shape
    l = num_col_input_ids.shape[1]
    cl = c * l
    n_pad = ((cl + 255) // 256) * 256

    ids = num_col_input_ids.reshape(cl).astype(jnp.int32)
    ids = jnp.pad(ids, (0, n_pad - cl))
    emb = _sc_gather(word_emb, ids, n_pad)

    mask = num_att_mask.reshape(cl, 1).astype(jnp.float32)
    p, q = pl.pallas_call(
        functools.partial(_pq_body, l),
        out_shape=(
            jax.ShapeDtypeStruct((c, H), jnp.float32),
            jax.ShapeDtypeStruct((1, H), jnp.float32),
        ),
    )(emb, mask, ln_w.reshape(1, H), ln_b.reshape(1, H),
      num_bias.reshape(1, H), align_W)

    x_pad = jnp.pad(x_num, ((0, 0), (0, H - c)))
    out = _sc_bc(x_pad, p, q, b, c)

    attention_mask = jnp.ones((b, c), dtype=jnp.float32)
    return (out, attention_mask)


# speedup vs baseline: 2.8398x; 2.8067x over previous
import functools

import jax
import jax.numpy as jnp
from jax import lax
from jax.experimental import pallas as pl
from jax.experimental.pallas import tpu as pltpu
from jax.experimental.pallas import tpu_sc as plsc

H = 128
EPS = 1e-05
BB = 128
NBUF = 4


def _sc_gather(table, idx_pad, n_pad):
    info = plsc.get_sparse_core_info()
    nw = info.num_cores * info.num_subcores
    per = n_pad // nw

    @functools.partial(
        pl.kernel,
        mesh=plsc.VectorSubcoreMesh(core_axis_name="c", subcore_axis_name="s"),
        out_type=jax.ShapeDtypeStruct((n_pad, H), jnp.float32),
        scratch_types=[
            pltpu.VMEM((per,), jnp.int32),
            pltpu.VMEM((per, H), jnp.float32),
            pltpu.SemaphoreType.DMA,
        ],
    )
    def gk(table_hbm, idx_hbm, out_hbm, idx_v, rows_v, sem):
        wid = lax.axis_index("s") * info.num_cores + lax.axis_index("c")
        base = wid * per
        pltpu.sync_copy(idx_hbm.at[pl.ds(base, per)], idx_v)
        pltpu.async_copy(table_hbm.at[idx_v], rows_v, sem).wait()
        pltpu.sync_copy(rows_v, out_hbm.at[pl.ds(base, per)])

    return gk(table, idx_pad)


def _main_body(l, emb_ref, mask_ref, lnw_ref, lnb_ref, bias_ref, w_ref,
               x_ref, out_ref, *rest):
    bufs, sems = rest[:NBUF], rest[NBUF:]
    cl = mask_ref.shape[0]

    e = emb_ref[0:cl, :]
    mu = jnp.mean(e, axis=1, keepdims=True)
    d = e - mu
    var = jnp.mean(d * d, axis=1, keepdims=True)
    ln = d * lax.rsqrt(var + EPS) * lnw_ref[:] + lnb_ref[:]
    m = mask_ref[:]
    s = jnp.sum((ln * m).reshape(cl // l, l, H), axis=1)
    cnt = jnp.sum(m.reshape(cl // l, l, 1), axis=1)
    col = s / cnt

    dn = (((1,), (1,)), ((), ()))
    p = lax.dot_general(col, w_ref[:], dn,
                        precision=lax.Precision.HIGHEST,
                        preferred_element_type=jnp.float32)
    q = lax.dot_general(bias_ref[:], w_ref[:], dn,
                        precision=lax.Precision.HIGHEST,
                        preferred_element_type=jnp.float32)

    cb = bufs[0].shape[0]
    nch = x_ref.shape[0] // cb
    for j in range(nch):
        r = j % NBUF
        if j >= NBUF:
            pltpu.make_async_copy(
                bufs[r], out_ref.at[pl.ds((j - NBUF) * cb, cb)],
                sems[r]).wait()
        x = x_ref[pl.ds(j * cb, cb), :]
        bufs[r][:] = x[:, :, None] * p + q
        pltpu.make_async_copy(
            bufs[r], out_ref.at[pl.ds(j * cb, cb)], sems[r]).start()
    for j in range(max(nch - NBUF, 0), nch):
        r = j % NBUF
        pltpu.make_async_copy(
            bufs[r], out_ref.at[pl.ds(j * cb, cb)], sems[r]).wait()


def kernel(x_num, num_col_input_ids, num_att_mask, word_emb, ln_w, ln_b, num_bias, align_W):
    b, c = x_num.shape
    l = num_col_input_ids.shape[1]
    cl = c * l
    n_pad = ((cl + 255) // 256) * 256

    ids = num_col_input_ids.reshape(cl).astype(jnp.int32)
    ids = jnp.pad(ids, (0, n_pad - cl))
    emb = _sc_gather(word_emb, ids, n_pad)

    mask = num_att_mask.reshape(cl, 1).astype(jnp.float32)
    out = pl.pallas_call(
        functools.partial(_main_body, l),
        out_specs=pl.BlockSpec(memory_space=pl.ANY),
        out_shape=jax.ShapeDtypeStruct((b, c, H), jnp.float32),
        scratch_shapes=(
            [pltpu.VMEM((BB, c, H), jnp.float32) for _ in range(NBUF)]
            + [pltpu.SemaphoreType.DMA for _ in range(NBUF)]
        ),
    )(emb, mask, ln_w.reshape(1, H), ln_b.reshape(1, H),
      num_bias.reshape(1, H), align_W, x_num)

    attention_mask = jnp.ones((b, c), dtype=jnp.float32)
    return (out, attention_mask)
